# baseline (speedup 1.0000x reference)
import functools

import jax
import jax.numpy as jnp
from jax import lax
from jax.experimental import pallas as pl
from jax.experimental.pallas import tpu as pltpu
from jax.experimental.pallas import tpu_sc as plsc

N = 10000
E = 320000
D = 128
H = 128
O = 64
G = 64

NPAD = 10112
ROWS_PER_TILE = 632
CH = 128
E_PAD = 327680
ECR = E_PAD // CH
ACH = 128
AGG_CH_PER_WORKER = E_PAD // (32 * ACH)
AGG_GRP = 8
DEG_CR_PER_WORKER = ECR // 32

_MESH = dict(core_axis_name="c", subcore_axis_name="s")
_F32 = jnp.float32


def _sc_degree(dst2d):
    @functools.partial(
        pl.kernel,
        out_type=jax.ShapeDtypeStruct((2, NPAD, 16), _F32),
        mesh=plsc.VectorSubcoreMesh(**_MESH),
        scratch_types=[
            pltpu.VMEM_SHARED((NPAD, 16), _F32),
            pltpu.VMEM((DEG_CR_PER_WORKER, CH), jnp.int32),
            pltpu.VMEM((ROWS_PER_TILE, 16), _F32),
            pltpu.VMEM((CH, 16), _F32),
        ],
    )
    def deg_kernel(dst_hbm, out_hbm, acc_sh, idx_v, zb_v, ones_v):
        c = lax.axis_index("c")
        s = lax.axis_index("s")
        w = c * 16 + s
        rs = pl.ds(s * ROWS_PER_TILE, ROWS_PER_TILE)

        @pl.loop(0, ROWS_PER_TILE)
        def _(i):
            zb_v[i, :] = jnp.zeros((16,), _F32)

        @pl.loop(0, CH)
        def _(i):
            ones_v[i, :] = jnp.ones((16,), _F32)

        pltpu.sync_copy(zb_v, acc_sh.at[rs, :])
        pltpu.sync_copy(dst_hbm.at[w], idx_v)
        plsc.subcore_barrier()

        @pl.loop(0, DEG_CR_PER_WORKER)
        def _(j):
            pltpu.sync_copy(ones_v, acc_sh.at[idx_v.at[j]], add=True)

        plsc.subcore_barrier()
        pltpu.sync_copy(acc_sh.at[rs, :], out_hbm.at[c, rs, :])

    return deg_kernel(dst2d)


def _sc_aggregate(hs, src3d, dst3d):
    @functools.partial(
        pl.kernel,
        out_type=jax.ShapeDtypeStruct((2, NPAD, H), _F32),
        mesh=plsc.VectorSubcoreMesh(**_MESH),
        scratch_types=[
            pltpu.VMEM_SHARED((NPAD, H), _F32),
            pltpu.VMEM((AGG_GRP, ACH), jnp.int32),
            pltpu.VMEM((AGG_GRP, ACH), jnp.int32),
            pltpu.VMEM((AGG_GRP, ACH), jnp.int32),
            pltpu.VMEM((AGG_GRP, ACH), jnp.int32),
            pltpu.VMEM((ACH, H), _F32),
            pltpu.VMEM((ACH, H), _F32),
            pltpu.VMEM((8, H), _F32),
            pltpu.SemaphoreType.DMA,
            pltpu.SemaphoreType.DMA,
            pltpu.SemaphoreType.DMA,
            pltpu.SemaphoreType.DMA,
            pltpu.SemaphoreType.DMA,
            pltpu.SemaphoreType.DMA,
        ],
    )
    def agg_kernel(hs_hbm, src_hbm, dst_hbm, out_hbm, acc_sh,
                   sia, dia, sib, dib, r0, r1, zb_v,
                   sg0, sg1, ss0, ss1, sem_ia, sem_ib):
        c = lax.axis_index("c")
        s = lax.axis_index("s")
        w = c * 16 + s
        rs = pl.ds(s * ROWS_PER_TILE, ROWS_PER_TILE)
        rows = (r0, r1)
        sem_g = (sg0, sg1)
        sem_s = (ss0, ss1)
        idx = ((sia, dia, sem_ia), (sib, dib, sem_ib))

        @pl.loop(0, 8)
        def _(i):
            @pl.loop(0, H // 16)
            def _(q):
                zb_v[i, pl.ds(q * 16, 16)] = jnp.zeros((16,), _F32)

        @pl.loop(0, ROWS_PER_TILE // 8)
        def _(i):
            pltpu.sync_copy(zb_v, acc_sh.at[pl.ds(s * ROWS_PER_TILE + i * 8, 8), :])

        plsc.subcore_barrier()

        def fire_gather(si, r, j):
            pltpu.async_copy(hs_hbm.at[si.at[r]], rows[j], sem_g[j])

        def fire_idx_load(g, si, di, sem):
            gs = pl.ds(g * AGG_GRP, AGG_GRP)
            pltpu.async_copy(src_hbm.at[w, gs, :], si, sem)
            pltpu.async_copy(dst_hbm.at[w, gs, :], di, sem)

        def wait_rows(sem):
            pltpu.make_async_copy(hs_hbm.at[pl.ds(0, ACH), :], r0, sem).wait()

        def wait_idx(sem):
            pltpu.make_async_copy(src_hbm.at[w, pl.ds(0, AGG_GRP), :], sia, sem).wait()
            pltpu.make_async_copy(src_hbm.at[w, pl.ds(0, AGG_GRP), :], dia, sem).wait()

        pltpu.sync_copy(src_hbm.at[w, pl.ds(0, AGG_GRP), :], sia)
        pltpu.sync_copy(dst_hbm.at[w, pl.ds(0, AGG_GRP), :], dia)
        fire_idx_load(1, sib, dib, sem_ib)
        fire_gather(sia, 0, 0)

        nk = AGG_CH_PER_WORKER // (2 * AGG_GRP)

        @pl.loop(0, nk)
        def _(k):
            for half in range(2):
                si_c, di_c, _ = idx[half]
                si_n, di_n, sem_n = idx[1 - half]
                for q in range(AGG_GRP):
                    i = q % 2
                    j = 1 - i
                    wait_rows(sem_g[i])
                    pltpu.async_copy(rows[i], acc_sh.at[di_c.at[q]],
                                     sem_s[i], add=True)
                    if half == 0 and q == 0:
                        @pl.when(k > 0)
                        def _():
                            wait_rows(sem_s[j])
                        fire_gather(si_c, q + 1, j)
                    elif q < AGG_GRP - 1:
                        wait_rows(sem_s[j])
                        fire_gather(si_c, q + 1, j)
                    else:
                        if half == 0:
                            wait_idx(sem_n)
                            wait_rows(sem_s[j])
                            fire_gather(si_n, 0, j)
                            @pl.when(k < nk - 1)
                            def _():
                                fire_idx_load(2 * k + 2, si_c, di_c, sem_ia)
                        else:
                            @pl.when(k < nk - 1)
                            def _():
                                wait_idx(sem_ia)
                                wait_rows(sem_s[j])
                                fire_gather(si_n, 0, j)
                                fire_idx_load(2 * k + 3, si_c, di_c, sem_ib)

        wait_rows(sem_s[0])
        wait_rows(sem_s[1])
        plsc.subcore_barrier()
        pltpu.sync_copy(acc_sh.at[rs, :], out_hbm.at[c, rs, :])

    return agg_kernel(hs, src3d, dst3d)


def _dinv_from_parts(dp):
    deg = dp[0] + dp[1] + 1.0
    return lax.rsqrt(deg)[:, 0:1]


def _pad_store(o_ref, hs):
    o_ref[:N, :] = hs
    o_ref[N:, :] = jnp.zeros((NPAD - N, H), _F32)


def _tc_scale_mm(x, W, degp):
    def body(x_ref, w_ref, dp_ref, o_ref):
        dinv = _dinv_from_parts(dp_ref[...])
        xw = lax.dot_general(
            x_ref[...], w_ref[...], (((1,), (0,)), ((), ())),
            precision=lax.Precision.HIGHEST, preferred_element_type=_F32)
        _pad_store(o_ref, xw * dinv[:N])

    return pl.pallas_call(
        body, out_shape=jax.ShapeDtypeStruct((NPAD, H), _F32))(x, W, degp)


def _tc_layer(agg, hs, degp, b, W):
    def body(agg_ref, hs_ref, dp_ref, b_ref, w_ref, o_ref):
        dinv = _dinv_from_parts(dp_ref[...])
        tot = agg_ref[0, :N, :] + agg_ref[1, :N, :] + hs_ref[:N, :]
        h = jnp.maximum(tot * dinv[:N] + b_ref[...], 0.0)
        hs2 = lax.dot_general(
            h, w_ref[...], (((1,), (0,)), ((), ())),
            precision=lax.Precision.HIGHEST, preferred_element_type=_F32)
        _pad_store(o_ref, hs2 * dinv[:N])

    return pl.pallas_call(
        body, out_shape=jax.ShapeDtypeStruct((NPAD, H), _F32))(agg, hs, degp, b, W)


def _tc_head(agg, hs, degp, b, batch, Wm, bm):
    def body(agg_ref, hs_ref, dp_ref, b_ref, bt_ref, wm_ref, bm_ref, out_ref, gx_ref):
        dinv = _dinv_from_parts(dp_ref[...])
        tot = agg_ref[0, :N, :] + agg_ref[1, :N, :] + hs_ref[:N, :]
        h = tot * dinv[:N] + b_ref[...]
        onehot = (bt_ref[...][:, None]
                  == lax.broadcasted_iota(jnp.int32, (N, G), 1)).astype(_F32)
        counts = jnp.sum(onehot, axis=0)
        pooled = lax.dot_general(
            onehot, h, (((0,), (0,)), ((), ())),
            precision=lax.Precision.HIGHEST, preferred_element_type=_F32)
        gx = pooled / jnp.maximum(counts, 1.0)[:, None]
        out = lax.dot_general(
            gx, wm_ref[...], (((1,), (0,)), ((), ())),
            precision=lax.Precision.HIGHEST, preferred_element_type=_F32)
        out_ref[...] = out + bm_ref[...]
        gx_ref[...] = gx

    return pl.pallas_call(
        body,
        out_shape=(jax.ShapeDtypeStruct((G, O), _F32),
                   jax.ShapeDtypeStruct((G, H), _F32)),
    )(agg, hs, degp, b, batch, Wm, bm)


def kernel(x, edge_index, batch, W1, b1, W2, b2, Wm, bm):
    pad = N + (jnp.arange(E_PAD - E, dtype=jnp.int32) % (NPAD - N))
    src_p = jnp.concatenate([edge_index[0], pad])
    dst_p = jnp.concatenate([edge_index[1], pad])
    src3d = src_p.reshape(32, AGG_CH_PER_WORKER, ACH)
    dst3d = dst_p.reshape(32, AGG_CH_PER_WORKER, ACH)
    dst_deg = dst_p.reshape(32, DEG_CR_PER_WORKER, CH)

    degp = _sc_degree(dst_deg)
    hs1 = _tc_scale_mm(x, W1, degp)
    agg1 = _sc_aggregate(hs1, src3d, dst3d)
    hs2 = _tc_layer(agg1, hs1, degp, b1, W2)
    agg2 = _sc_aggregate(hs2, src3d, dst3d)
    return _tc_head(agg2, hs2, degp, b2, batch, Wm, bm)
